# Initial kernel scaffold; baseline (speedup 1.0000x reference)
import jax
import jax.numpy as jnp
from jax.experimental import pallas as pl


def kernel(Img, Tform):
    raise NotImplementedError("write your pallas kernel here")


# speedup vs baseline: 4.0740x; 4.0740x over previous
import functools

import jax
import jax.numpy as jnp
from jax import lax
from jax.experimental import pallas as pl
from jax.experimental.pallas import tpu as pltpu
from jax.experimental.pallas import tpu_sc as plsc

H = W = D = 128
HP = 129
V = HP * HP * HP
B = 4
N = B * H * W * D
NW = 32
P = N // NW
C = 4096
CH = C // 128
NCHUNK = P // C
OCT_TAIL = HP * HP + HP + 2


def _bf(t):
    return lax.convert_element_type(
        lax.convert_element_type(t, jnp.bfloat16), jnp.float32)


def _coord(tf_ref, b, gxb, gyb, gzb, row):
    p0 = _bf(tf_ref[b, row * 3]) * gxb
    p1 = _bf(tf_ref[b, row * 3 + 1]) * gyb
    p2 = _bf(tf_ref[b, row * 3 + 2]) * gzb
    x_s = ((p0 + p1) + p2) + tf_ref[b, 9 + row]
    return 0.5 * (x_s + 1.0) * jnp.float32(127.0)


def _idx_body(tf_ref, idx_ref):
    b = pl.program_id(0)
    i = pl.program_id(1)
    scale = jnp.float32(2.0 / 127.0)
    gx = _bf(i.astype(jnp.float32) * scale - 1.0)
    gy = _bf(lax.broadcasted_iota(jnp.int32, (H, D), 0).astype(jnp.float32) * scale - 1.0)
    gz = _bf(lax.broadcasted_iota(jnp.int32, (H, D), 1).astype(jnp.float32) * scale - 1.0)
    x = _coord(tf_ref, b, gx, gy, gz, 0)
    y = _coord(tf_ref, b, gx, gy, gz, 1)
    z = _coord(tf_ref, b, gx, gy, gz, 2)
    x0 = jnp.clip(jnp.floor(x).astype(jnp.int32), 0, 128)
    y0 = jnp.clip(jnp.floor(y).astype(jnp.int32), 0, 128)
    z0 = jnp.clip(jnp.floor(z).astype(jnp.int32), 0, 128)
    idx_ref[0, 0] = (x0 * HP + y0) * HP + z0 + b * V


def _indices(Tform):
    return pl.pallas_call(
        _idx_body,
        grid=(B, W),
        in_specs=[pl.BlockSpec(memory_space=pltpu.SMEM)],
        out_specs=pl.BlockSpec((1, 1, H, D), lambda b, i: (b, i, 0, 0)),
        out_shape=jax.ShapeDtypeStruct((B, H, W, D), jnp.int32),
    )(Tform)


def _sc_gather_body(oct_hbm, idx_hbm, vals_hbm, idx_v, rows_v, sem):
    wid = lax.axis_index("s") * 2 + lax.axis_index("c")
    base0 = wid * P

    def chunk_body(ci, carry):
        base = pl.multiple_of(base0 + ci * C, C)
        rbase = pl.multiple_of(base // 128, CH)
        pltpu.sync_copy(idx_hbm.at[pl.ds(rbase, CH)], idx_v)

        def fire(r, c2):
            pltpu.async_copy(oct_hbm.at[idx_v.at[r]],
                             rows_v.at[pl.ds(r * 128, 128)], sem)
            return c2

        lax.fori_loop(0, CH, fire, 0)
        pltpu.make_async_copy(oct_hbm.at[pl.ds(0, C)], rows_v, sem).wait()
        pltpu.sync_copy(rows_v, vals_hbm.at[pl.ds(base, C)])
        return carry

    lax.fori_loop(0, NCHUNK, chunk_body, 0)


@functools.cache
def _sc_gather():
    mesh = plsc.VectorSubcoreMesh(core_axis_name="c", subcore_axis_name="s",
                                  num_cores=2, num_subcores=16)
    return pl.kernel(
        _sc_gather_body,
        out_type=jax.ShapeDtypeStruct((N, 8), jnp.float32),
        mesh=mesh,
        scratch_types=[
            pltpu.VMEM((CH, 128), jnp.int32),
            pltpu.VMEM((C, 8), jnp.float32),
            pltpu.SemaphoreType.DMA,
        ],
        compiler_params=pltpu.CompilerParams(use_tc_tiling_on_sc=False),
    )


def _combine_body(tf_ref, vals_ref, out_ref):
    pid = pl.program_id(0)
    b = pid // H
    i = pid % H
    scale = jnp.float32(2.0 / 127.0)
    gx = _bf(i.astype(jnp.float32) * scale - 1.0)
    lane = lax.broadcasted_iota(jnp.int32, (1, W, D * 8), 2)
    k = lax.shift_right_logical(lane, 3)
    corner = lane & 7
    gy = _bf(lax.broadcasted_iota(jnp.int32, (1, W, D * 8), 1).astype(jnp.float32) * scale - 1.0)
    gz = _bf(k.astype(jnp.float32) * scale - 1.0)
    x = _coord(tf_ref, b, gx, gy, gz, 0)
    y = _coord(tf_ref, b, gx, gy, gz, 1)
    z = _coord(tf_ref, b, gx, gy, gz, 2)
    fone = jnp.float32(1.0)
    fzero = jnp.float32(0.0)

    def axis_w(t, bit):
        f = t - jnp.floor(t)
        wsel = jnp.where(bit == 1, f, fone - f)
        return jnp.where((t >= 0.0) & (t < 128.0), wsel, fzero)

    wx = axis_w(x, lax.shift_right_logical(corner, 2) & 1)
    wy = axis_w(y, lax.shift_right_logical(corner, 1) & 1)
    wz = axis_w(z, corner & 1)
    prod = vals_ref[...] * (wx * wy * wz)
    out_ref[...] = prod.reshape(1, W, D, 8).sum(axis=-1)


def _combine(Tform, vals):
    return pl.pallas_call(
        _combine_body,
        grid=(B * H,),
        in_specs=[
            pl.BlockSpec(memory_space=pltpu.SMEM),
            pl.BlockSpec((1, W, D * 8), lambda p: (p, 0, 0)),
        ],
        out_specs=pl.BlockSpec((1, W, D), lambda p: (p, 0, 0)),
        out_shape=jax.ShapeDtypeStruct((B * H, W, D), jnp.float32),
    )(Tform, vals.reshape(B * H, W, D * 8))


def _build_oct(Img):
    imgp = jnp.pad(Img[..., 0], ((0, 0), (0, 1), (0, 1), (0, 1)))
    flat = imgp.reshape(-1)
    flat_ext = jnp.concatenate([flat, jnp.zeros((OCT_TAIL,), jnp.float32)])
    offs = (0, 1, HP, HP + 1, HP * HP, HP * HP + 1, HP * HP + HP,
            HP * HP + HP + 1)
    return jnp.stack(
        [lax.dynamic_slice_in_dim(flat_ext, o, B * V) for o in offs], axis=1)


def kernel(Img, Tform):
    idx = _indices(Tform)
    octt = _build_oct(Img)
    vals = _sc_gather()(octt, idx.reshape(N // 128, 128))
    out = _combine(Tform, vals)
    return out.reshape(B, H, W, D, 1)


# speedup vs baseline: 10.8549x; 2.6644x over previous
import functools

import jax
import jax.numpy as jnp
from jax import lax
from jax.experimental import pallas as pl
from jax.experimental.pallas import tpu as pltpu
from jax.experimental.pallas import tpu_sc as plsc

H = W = D = 128
V = H * W * D
B = 4
N = B * V
NW = 32
TAIL = 16520
P = N // NW
C = 4096
CH = C // 128
NCHUNK = P // C
CW = 8192
NBCHUNK = P // CW


def _bf(t):
    return lax.convert_element_type(
        lax.convert_element_type(t, jnp.bfloat16), jnp.float32)


def _coord(tf_ref, b, gxb, gyb, gzb, row):
    p0 = _bf(tf_ref[b, row * 3]) * gxb
    p1 = _bf(tf_ref[b, row * 3 + 1]) * gyb
    p2 = _bf(tf_ref[b, row * 3 + 2]) * gzb
    x_s = ((p0 + p1) + p2) + tf_ref[b, 9 + row]
    return 0.5 * (x_s + 1.0) * jnp.float32(127.0)


def _idx_body(tf_ref, idx_ref):
    b = pl.program_id(0)
    i = pl.program_id(1)
    scale = jnp.float32(2.0 / 127.0)
    gx = _bf(i.astype(jnp.float32) * scale - 1.0)
    gy = _bf(lax.broadcasted_iota(jnp.int32, (H, D), 0).astype(jnp.float32) * scale - 1.0)
    gz = _bf(lax.broadcasted_iota(jnp.int32, (H, D), 1).astype(jnp.float32) * scale - 1.0)
    x = _coord(tf_ref, b, gx, gy, gz, 0)
    y = _coord(tf_ref, b, gx, gy, gz, 1)
    z = _coord(tf_ref, b, gx, gy, gz, 2)
    x0 = jnp.clip(jnp.floor(x).astype(jnp.int32), 0, 127)
    y0 = jnp.clip(jnp.floor(y).astype(jnp.int32), 0, 127)
    z0 = jnp.clip(jnp.floor(z).astype(jnp.int32), 0, 127)
    idx_ref[0, 0] = ((b * H + x0) * W + y0) * D + z0


def _indices(Tform):
    return pl.pallas_call(
        _idx_body,
        grid=(B, W),
        in_specs=[pl.BlockSpec(memory_space=pltpu.SMEM)],
        out_specs=pl.BlockSpec((1, 1, H, D), lambda b, i: (b, i, 0, 0)),
        out_shape=jax.ShapeDtypeStruct((B, H, W, D), jnp.int32),
    )(Tform)


def _oct_build_body(flat_hbm, oct_hbm, win_v, out_v, sem):
    wid = lax.axis_index("s") * 2 + lax.axis_index("c")
    base0 = wid * P
    lanes = lax.broadcasted_iota(jnp.int32, (16,), 0)
    pat = ((lanes >> 3) + (lanes & 1) + ((lanes >> 1) & 1) * D
           + ((lanes >> 2) & 1) * (W * D))

    def chunk_body(ci, carry):
        base = pl.multiple_of(base0 + ci * CW, CW)
        pltpu.sync_copy(flat_hbm.at[pl.ds(base, CW + TAIL)], win_v)

        def grp(g, c2):
            out_v[pl.ds(g * 16, 16)] = plsc.load_gather(win_v, [pat + g * 2])
            return c2

        lax.fori_loop(0, CW * 8 // 16, grp, 0)
        pltpu.sync_copy(out_v, oct_hbm.at[pl.ds(base * 8, CW * 8)])
        return carry

    lax.fori_loop(0, NBCHUNK, chunk_body, 0)


def _sc_gather_body(oct_hbm, idx_hbm, vals_hbm, idx_v, rows_v, out_v, sem):
    wid = lax.axis_index("s") * 2 + lax.axis_index("c")
    base0 = wid * P
    lanes = lax.broadcasted_iota(jnp.int32, (16,), 0)

    def chunk_body(ci, carry):
        base = pl.multiple_of(base0 + ci * C, C)
        rbase = pl.multiple_of(base // 128, CH)
        pltpu.sync_copy(idx_hbm.at[pl.ds(rbase, CH)], idx_v)

        def fire(r, c2):
            pltpu.async_copy(oct_hbm.at[idx_v.at[r]],
                             rows_v.at[pl.ds(r * 128, 128)], sem)
            return c2

        lax.fori_loop(0, CH, fire, 0)
        pltpu.make_async_copy(oct_hbm.at[pl.ds(0, C)], rows_v, sem).wait()

        def grp(t, c2):
            pid = (t >> 6) * 128 + (t & 7) * 16 + lanes
            crn = jnp.broadcast_to((t >> 3) & 7, (16,))
            out_v[pl.ds(t * 16, 16)] = plsc.load_gather(rows_v, [pid, crn])
            return c2

        lax.fori_loop(0, C * 8 // 16, grp, 0)
        pltpu.sync_copy(out_v, vals_hbm.at[pl.ds(base * 8, C * 8)])
        return carry

    lax.fori_loop(0, NCHUNK, chunk_body, 0)


@functools.cache
def _sc_kernels():
    mesh = plsc.VectorSubcoreMesh(core_axis_name="c", subcore_axis_name="s",
                                  num_cores=2, num_subcores=16)
    params = pltpu.CompilerParams(use_tc_tiling_on_sc=False,
                                  needs_layout_passes=False)
    build = pl.kernel(
        _oct_build_body,
        out_type=jax.ShapeDtypeStruct((N * 8,), jnp.float32),
        mesh=mesh,
        scratch_types=[
            pltpu.VMEM((CW + TAIL,), jnp.float32),
            pltpu.VMEM((CW * 8,), jnp.float32),
            pltpu.SemaphoreType.DMA,
        ],
        compiler_params=params,
    )
    gather = pl.kernel(
        _sc_gather_body,
        out_type=jax.ShapeDtypeStruct((N * 8,), jnp.float32),
        mesh=mesh,
        scratch_types=[
            pltpu.VMEM((CH, 128), jnp.int32),
            pltpu.VMEM((C, 8), jnp.float32),
            pltpu.VMEM((C * 8,), jnp.float32),
            pltpu.SemaphoreType.DMA,
        ],
        compiler_params=params,
    )
    return build, gather


def _combine_body(tf_ref, vals_ref, sum8_ref, out_ref):
    pid = pl.program_id(0)
    b = pid // H
    i = pid % H
    scale = jnp.float32(2.0 / 127.0)
    gx = _bf(i.astype(jnp.float32) * scale - 1.0)
    r = lax.broadcasted_iota(jnp.int32, (1, 1024, 128), 1)
    k = lax.broadcasted_iota(jnp.int32, (1, 1024, 128), 2)
    j = r >> 3
    corner = r & 7
    gy = _bf(j.astype(jnp.float32) * scale - 1.0)
    gz = _bf(k.astype(jnp.float32) * scale - 1.0)
    x = _coord(tf_ref, b, gx, gy, gz, 0)
    y = _coord(tf_ref, b, gx, gy, gz, 1)
    z = _coord(tf_ref, b, gx, gy, gz, 2)
    fone = jnp.float32(1.0)
    fzero = jnp.float32(0.0)

    def axis_w(t, bit):
        f = t - jnp.floor(t)
        wsel = jnp.where(bit == 1, f, fone - f)
        lim = jnp.float32(128.0) - bit.astype(jnp.float32)
        return jnp.where((t >= 0.0) & (t < lim), wsel, fzero)

    wx = axis_w(x, lax.shift_right_logical(corner, 2) & 1)
    wy = axis_w(y, lax.shift_right_logical(corner, 1) & 1)
    wz = axis_w(z, corner & 1)
    prod = (vals_ref[...] * (wx * wy * wz)).reshape(1024, 128)
    s = lax.dot_general(sum8_ref[...], prod, (((1,), (0,)), ((), ())),
                        precision=lax.Precision.HIGHEST)
    out_ref[...] = s.reshape(1, W, D)


def _combine(Tform, vals):
    sum8 = jnp.repeat(jnp.eye(W, dtype=jnp.float32), 8, axis=1)
    return pl.pallas_call(
        _combine_body,
        grid=(B * H,),
        in_specs=[
            pl.BlockSpec(memory_space=pltpu.SMEM),
            pl.BlockSpec((1, 1024, 128), lambda p: (p, 0, 0)),
            pl.BlockSpec((W, 1024), lambda p: (0, 0)),
        ],
        out_specs=pl.BlockSpec((1, W, D), lambda p: (p, 0, 0)),
        out_shape=jax.ShapeDtypeStruct((B * H, W, D), jnp.float32),
    )(Tform, vals.reshape(B * H, 1024, 128), sum8)


def kernel(Img, Tform):
    idx = _indices(Tform)
    build, gather = _sc_kernels()
    flat_ext = jnp.concatenate(
        [Img.reshape(N), jnp.zeros((TAIL,), jnp.float32)])
    octt = build(flat_ext).reshape(N, 8)
    vals = gather(octt, idx.reshape(N // 128, 128))
    out = _combine(Tform, vals)
    return out.reshape(B, H, W, D, 1)
